# baseline (speedup 1.0000x reference)
import functools

import jax
import jax.numpy as jnp
from jax import lax
from jax.experimental import pallas as pl
from jax.experimental.pallas import tpu as pltpu
from jax.experimental.pallas import tpu_sc as plsc

NUM_CORES = 2
NUM_SUBCORES = 16
VPAD = 1024


def _softmax_table_body(table_ref, w_ref, b_ref, out_ref):
    logits = jnp.dot(table_ref[...], w_ref[...],
                     preferred_element_type=jnp.float32)
    logits = logits + b_ref[...]
    m = jnp.max(logits, axis=-1, keepdims=True)
    e = jnp.exp(logits - m)
    out_ref[...] = e / jnp.sum(e, axis=-1, keepdims=True)


def _compute_prob_table(table, W, b):
    V = W.shape[1]
    W_pad = jnp.pad(W, ((0, 0), (0, VPAD - V)))
    b_pad = jnp.pad(b, (0, VPAD - V), constant_values=-1e30)
    return pl.pallas_call(
        _softmax_table_body,
        out_shape=jax.ShapeDtypeStruct((table.shape[0], VPAD), jnp.float32),
    )(table, W_pad, b_pad.reshape(1, VPAD))


@functools.lru_cache(maxsize=None)
def _make_row_gather(n_tokens, V, chunk):
    nw = NUM_CORES * NUM_SUBCORES
    total_chunks = n_tokens // chunk
    n_chunks = total_chunks // nw
    assert n_tokens % (chunk * nw) == 0 and n_chunks % 2 == 0 and n_chunks >= 4
    assert (chunk * V * 4) % 64 == 0

    mesh = plsc.VectorSubcoreMesh(core_axis_name="c", subcore_axis_name="s")

    @functools.partial(
        pl.kernel,
        mesh=mesh,
        compiler_params=pltpu.CompilerParams(use_tc_tiling_on_sc=False),
        out_type=jax.ShapeDtypeStruct((total_chunks, chunk, V), jnp.float32),
        scratch_types=[
            pltpu.VMEM((n_chunks, chunk), jnp.int32),
            pltpu.VMEM((2, chunk, VPAD), jnp.float32),
            pltpu.SemaphoreType.DMA,
            pltpu.SemaphoreType.DMA,
            pltpu.SemaphoreType.DMA,
            pltpu.SemaphoreType.DMA,
        ],
    )
    def gather_kernel(prob_hbm, idx_hbm, out_hbm, idx_v, rows_v,
                      gsem0, gsem1, ssem0, ssem1):
        sid = lax.axis_index("s")
        wid = sid * NUM_CORES + lax.axis_index("c")
        base = wid * n_chunks

        pltpu.sync_copy(idx_hbm.at[pl.ds(base, n_chunks)], idx_v)

        gsem = (gsem0, gsem1)
        ssem = (ssem0, ssem1)

        def start_g(g, buf):
            pltpu.async_copy(prob_hbm.at[idx_v.at[g]], rows_v.at[buf],
                             gsem[buf])

        def wait_g(g, buf):
            pltpu.make_async_copy(prob_hbm.at[idx_v.at[g]], rows_v.at[buf],
                                  gsem[buf]).wait()

        def start_s(g, buf):
            pltpu.async_copy(rows_v.at[buf, :, pl.ds(0, V)],
                             out_hbm.at[base + g], ssem[buf])

        def wait_s(g, buf):
            pltpu.make_async_copy(rows_v.at[buf, :, pl.ds(0, V)],
                                  out_hbm.at[base + g], ssem[buf]).wait()

        start_g(0, 0)
        wait_g(0, 0)
        start_s(0, 0)
        start_g(1, 1)
        wait_g(1, 1)
        start_s(1, 1)
        wait_s(0, 0)
        start_g(2, 0)

        def round_body(i, _):
            g0 = 2 * i
            wait_g(g0, 0)
            start_s(g0, 0)
            wait_s(g0 - 1, 1)
            start_g(g0 + 1, 1)
            wait_g(g0 + 1, 1)
            start_s(g0 + 1, 1)
            wait_s(g0, 0)
            start_g(g0 + 2, 0)
            return 0

        lax.fori_loop(1, n_chunks // 2 - 1, round_body, 0)

        gl = n_chunks - 2
        wait_g(gl, 0)
        start_s(gl, 0)
        wait_s(gl - 1, 1)
        start_g(gl + 1, 1)
        wait_g(gl + 1, 1)
        start_s(gl + 1, 1)
        wait_s(gl, 0)
        wait_s(gl + 1, 1)

    return gather_kernel


def kernel(inputs, table, W, b):
    B, S = inputs.shape
    V = W.shape[1]
    chunk = 32
    prob = _compute_prob_table(table, W, b)
    idx = inputs.reshape(B * S // chunk, chunk)
    out = _make_row_gather(B * S, V, chunk)(prob, idx)
    return out.reshape(B, S, V)


# speedup vs baseline: 1.3354x; 1.3354x over previous
import functools

import jax
import jax.numpy as jnp
from jax import lax
from jax.experimental import pallas as pl
from jax.experimental.pallas import tpu as pltpu
from jax.experimental.pallas import tpu_sc as plsc

NUM_CORES = 2
NUM_SUBCORES = 16
VPAD = 1024


def _softmax_table_body(table_ref, w_ref, b_ref, out_ref):
    logits = jnp.dot(table_ref[...], w_ref[...],
                     preferred_element_type=jnp.float32)
    logits = logits + b_ref[...]
    m = jnp.max(logits, axis=-1, keepdims=True)
    e = jnp.exp(logits - m)
    out_ref[...] = e / jnp.sum(e, axis=-1, keepdims=True)


def _compute_prob_table(table, W, b):
    V = W.shape[1]
    W_pad = jnp.pad(W, ((0, 0), (0, VPAD - V)))
    b_pad = jnp.pad(b, (0, VPAD - V), constant_values=-1e30)
    return pl.pallas_call(
        _softmax_table_body,
        out_shape=jax.ShapeDtypeStruct((table.shape[0], VPAD), jnp.float32),
    )(table, W_pad, b_pad.reshape(1, VPAD))


def _tc_gather_body(idx_ref, p_ref, out_ref):
    t = idx_ref.shape[0]
    v = p_ref.shape[0]
    onehot = (idx_ref[...][:, None] ==
              lax.broadcasted_iota(jnp.int32, (t, v), 1))
    out_ref[...] = jnp.dot(onehot.astype(jnp.bfloat16), p_ref[...],
                           preferred_element_type=jnp.float32)


@functools.lru_cache(maxsize=None)
def _make_tc_gather(n_tokens, V, T):
    grid = (n_tokens // T,)
    return pl.pallas_call(
        _tc_gather_body,
        grid=grid,
        in_specs=[
            pl.BlockSpec((T,), lambda i: (i,)),
            pl.BlockSpec((V, V), lambda i: (0, 0)),
        ],
        out_specs=pl.BlockSpec((T, V), lambda i: (i, 0)),
        out_shape=jax.ShapeDtypeStruct((n_tokens, V), jnp.float32),
    )


@functools.lru_cache(maxsize=None)
def _make_row_gather(n_tokens, V, chunk):
    nw = NUM_CORES * NUM_SUBCORES
    total_chunks = n_tokens // chunk
    n_chunks = total_chunks // nw
    assert n_tokens % (chunk * nw) == 0 and n_chunks % 2 == 0 and n_chunks >= 4
    assert (chunk * V * 4) % 64 == 0

    mesh = plsc.VectorSubcoreMesh(core_axis_name="c", subcore_axis_name="s")

    @functools.partial(
        pl.kernel,
        mesh=mesh,
        compiler_params=pltpu.CompilerParams(use_tc_tiling_on_sc=False),
        out_type=jax.ShapeDtypeStruct((total_chunks, chunk, V), jnp.float32),
        scratch_types=[
            pltpu.VMEM((n_chunks, chunk), jnp.int32),
            pltpu.VMEM((2, chunk, VPAD), jnp.float32),
            pltpu.SemaphoreType.DMA,
            pltpu.SemaphoreType.DMA,
            pltpu.SemaphoreType.DMA,
            pltpu.SemaphoreType.DMA,
        ],
    )
    def gather_kernel(prob_hbm, idx_hbm, out_hbm, idx_v, rows_v,
                      gsem0, gsem1, ssem0, ssem1):
        sid = lax.axis_index("s")
        wid = sid * NUM_CORES + lax.axis_index("c")
        base = wid * n_chunks

        pltpu.sync_copy(idx_hbm.at[pl.ds(base, n_chunks)], idx_v)

        gsem = (gsem0, gsem1)
        ssem = (ssem0, ssem1)

        def start_g(g, buf):
            pltpu.async_copy(prob_hbm.at[idx_v.at[g]], rows_v.at[buf],
                             gsem[buf])

        def wait_g(g, buf):
            pltpu.make_async_copy(prob_hbm.at[idx_v.at[g]], rows_v.at[buf],
                                  gsem[buf]).wait()

        def start_s(g, buf):
            pltpu.async_copy(rows_v.at[buf, :, pl.ds(0, V)],
                             out_hbm.at[base + g], ssem[buf])

        def wait_s(g, buf):
            pltpu.make_async_copy(rows_v.at[buf, :, pl.ds(0, V)],
                                  out_hbm.at[base + g], ssem[buf]).wait()

        start_g(0, 0)
        wait_g(0, 0)
        start_s(0, 0)
        start_g(1, 1)
        wait_g(1, 1)
        start_s(1, 1)
        wait_s(0, 0)
        start_g(2, 0)

        def round_body(i, _):
            g0 = 2 * i
            wait_g(g0, 0)
            start_s(g0, 0)
            wait_s(g0 - 1, 1)
            start_g(g0 + 1, 1)
            wait_g(g0 + 1, 1)
            start_s(g0 + 1, 1)
            wait_s(g0, 0)
            start_g(g0 + 2, 0)
            return 0

        lax.fori_loop(1, n_chunks // 2 - 1, round_body, 0)

        gl = n_chunks - 2
        wait_g(gl, 0)
        start_s(gl, 0)
        wait_s(gl - 1, 1)
        start_g(gl + 1, 1)
        wait_g(gl + 1, 1)
        start_s(gl + 1, 1)
        wait_s(gl, 0)
        wait_s(gl + 1, 1)

    return gather_kernel


def kernel(inputs, table, W, b):
    B, S = inputs.shape
    V = W.shape[1]
    prob = _compute_prob_table(table, W, b)
    p_bf16 = prob[:, :V].astype(jnp.bfloat16)
    out = _make_tc_gather(B * S, V, 256)(inputs.reshape(B * S), p_bf16)
    return out.reshape(B, S, V)
